# Initial kernel scaffold; baseline (speedup 1.0000x reference)
import jax
import jax.numpy as jnp
from jax.experimental import pallas as pl


def kernel(scaled_atomic_energy, batch, modal_type, atom_type, shift, scale):
    raise NotImplementedError("write your pallas kernel here")


# speedup vs baseline: 52.6306x; 52.6306x over previous
import functools

import jax
import jax.numpy as jnp
from jax import lax
from jax.experimental import pallas as pl
from jax.experimental.pallas import tpu as pltpu
from jax.experimental.pallas import tpu_sc as plsc

N = 100000
G = 64
L = 16
NW = 32
CHUNK = 3120
MAIN = NW * CHUNK
TAIL = N - MAIN


def _sc_body(e_hbm, b_hbm, mt_hbm, at_hbm, sh_hbm, sc_hbm, out_hbm,
             e_v, b_v, at_v, o_v, mt_v, sh_v, sc_v):
    cid = lax.axis_index("c")
    sid = lax.axis_index("s")
    wid = sid * 2 + cid
    base = wid * CHUNK

    pltpu.sync_copy(mt_hbm, mt_v)
    pltpu.sync_copy(sh_hbm, sh_v)
    pltpu.sync_copy(sc_hbm, sc_v)

    pltpu.sync_copy(e_hbm.at[pl.ds(base, CHUNK)], e_v)
    pltpu.sync_copy(b_hbm.at[pl.ds(base, CHUNK)], b_v)
    pltpu.sync_copy(at_hbm.at[pl.ds(base, CHUNK)], at_v)

    def step(i, _):
        off = i * L
        b = b_v[pl.ds(off, L)]
        a = at_v[pl.ds(off, L)]
        m = plsc.load_gather(mt_v, [b])
        idx = m * 16 + a
        sh = plsc.load_gather(sh_v, [idx])
        sc = plsc.load_gather(sc_v, [idx])
        o_v[pl.ds(off, L)] = e_v[pl.ds(off, L)] * sc + sh
        return 0

    lax.fori_loop(0, CHUNK // L, step, 0)
    pltpu.sync_copy(o_v, out_hbm.at[pl.ds(base, CHUNK)])

    @pl.when(wid == NW - 1)
    def _tail():
        pltpu.sync_copy(e_hbm.at[pl.ds(MAIN, TAIL)], e_v.at[pl.ds(0, TAIL)])
        pltpu.sync_copy(b_hbm.at[pl.ds(MAIN, TAIL)], b_v.at[pl.ds(0, TAIL)])
        pltpu.sync_copy(at_hbm.at[pl.ds(MAIN, TAIL)], at_v.at[pl.ds(0, TAIL)])
        lax.fori_loop(0, TAIL // L, step, 0)
        pltpu.sync_copy(o_v.at[pl.ds(0, TAIL)], out_hbm.at[pl.ds(MAIN, TAIL)])


_sc_call = functools.partial(
    pl.kernel,
    mesh=plsc.VectorSubcoreMesh(core_axis_name="c", subcore_axis_name="s"),
    out_type=jax.ShapeDtypeStruct((N,), jnp.float32),
    compiler_params=pltpu.CompilerParams(needs_layout_passes=False),
    scratch_types=[
        pltpu.VMEM((CHUNK,), jnp.float32),
        pltpu.VMEM((CHUNK,), jnp.int32),
        pltpu.VMEM((CHUNK,), jnp.int32),
        pltpu.VMEM((CHUNK,), jnp.float32),
        pltpu.VMEM((G,), jnp.int32),
        pltpu.VMEM((G,), jnp.float32),
        pltpu.VMEM((G,), jnp.float32),
    ],
)(_sc_body)


def kernel(scaled_atomic_energy, batch, modal_type, atom_type, shift, scale):
    e = scaled_atomic_energy.reshape(-1)
    out = _sc_call(e, batch, modal_type, atom_type,
                   shift.reshape(-1), scale.reshape(-1))
    return out.reshape(-1, 1)


# speedup vs baseline: 59.8681x; 1.1375x over previous
import functools

import jax
import jax.numpy as jnp
from jax import lax
from jax.experimental import pallas as pl
from jax.experimental.pallas import tpu as pltpu
from jax.experimental.pallas import tpu_sc as plsc

N = 100000
G = 64
L = 16
NW = 32
CHUNK = 3120
MAIN = NW * CHUNK
TAIL = N - MAIN


def _sc_body(e_hbm, b_hbm, mt_hbm, at_hbm, sh_hbm, sc_hbm, out_hbm,
             e_v, b_v, at_v, o_v, mt_v, sh_v, sc_v, sem):
    cid = lax.axis_index("c")
    sid = lax.axis_index("s")
    wid = sid * 2 + cid
    base = wid * CHUNK

    cps = (
        pltpu.async_copy(mt_hbm, mt_v, sem),
        pltpu.async_copy(sh_hbm, sh_v, sem),
        pltpu.async_copy(sc_hbm, sc_v, sem),
        pltpu.async_copy(e_hbm.at[pl.ds(base, CHUNK)], e_v, sem),
        pltpu.async_copy(b_hbm.at[pl.ds(base, CHUNK)], b_v, sem),
        pltpu.async_copy(at_hbm.at[pl.ds(base, CHUNK)], at_v, sem),
    )
    for cp in cps:
        cp.wait()

    def step(i, _):
        off = i * L
        b = b_v[pl.ds(off, L)]
        a = at_v[pl.ds(off, L)]
        m = plsc.load_gather(mt_v, [b])
        idx = m * 16 + a
        sh = plsc.load_gather(sh_v, [idx])
        sc = plsc.load_gather(sc_v, [idx])
        o_v[pl.ds(off, L)] = e_v[pl.ds(off, L)] * sc + sh
        return 0

    @plsc.parallel_loop(0, CHUNK // L, 1, unroll=5)
    def _main(i):
        step(i, 0)

    pltpu.sync_copy(o_v, out_hbm.at[pl.ds(base, CHUNK)])

    @pl.when(wid == NW - 1)
    def _tail():
        pltpu.sync_copy(e_hbm.at[pl.ds(MAIN, TAIL)], e_v.at[pl.ds(0, TAIL)])
        pltpu.sync_copy(b_hbm.at[pl.ds(MAIN, TAIL)], b_v.at[pl.ds(0, TAIL)])
        pltpu.sync_copy(at_hbm.at[pl.ds(MAIN, TAIL)], at_v.at[pl.ds(0, TAIL)])
        lax.fori_loop(0, TAIL // L, step, 0)
        pltpu.sync_copy(o_v.at[pl.ds(0, TAIL)], out_hbm.at[pl.ds(MAIN, TAIL)])


_sc_call = functools.partial(
    pl.kernel,
    mesh=plsc.VectorSubcoreMesh(core_axis_name="c", subcore_axis_name="s"),
    out_type=jax.ShapeDtypeStruct((N,), jnp.float32),
    compiler_params=pltpu.CompilerParams(needs_layout_passes=False),
    scratch_types=[
        pltpu.VMEM((CHUNK,), jnp.float32),
        pltpu.VMEM((CHUNK,), jnp.int32),
        pltpu.VMEM((CHUNK,), jnp.int32),
        pltpu.VMEM((CHUNK,), jnp.float32),
        pltpu.VMEM((G,), jnp.int32),
        pltpu.VMEM((G,), jnp.float32),
        pltpu.VMEM((G,), jnp.float32),
        pltpu.SemaphoreType.DMA,
    ],
)(_sc_body)


def kernel(scaled_atomic_energy, batch, modal_type, atom_type, shift, scale):
    e = scaled_atomic_energy.reshape(-1)
    out = _sc_call(e, batch, modal_type, atom_type,
                   shift.reshape(-1), scale.reshape(-1))
    return out.reshape(-1, 1)


# speedup vs baseline: 61.8511x; 1.0331x over previous
import functools

import jax
import jax.numpy as jnp
from jax import lax
from jax.experimental import pallas as pl
from jax.experimental.pallas import tpu as pltpu
from jax.experimental.pallas import tpu_sc as plsc

N = 100000
G = 64
L = 16
NW = 32
CHUNK = 3120
MAIN = NW * CHUNK
TAIL = N - MAIN


def _sc_body(e_hbm, b_hbm, mt_hbm, at_hbm, sh_hbm, sc_hbm, out_hbm,
             e_v, b_v, at_v, o_v, mt_v, sh_v, sc_v,
             et_v, bt_v, att_v, ot_v, sem):
    cid = lax.axis_index("c")
    sid = lax.axis_index("s")
    wid = sid * 2 + cid
    base = wid * CHUNK
    is_last = wid == NW - 1

    cps = (
        pltpu.async_copy(mt_hbm, mt_v, sem),
        pltpu.async_copy(sh_hbm, sh_v, sem),
        pltpu.async_copy(sc_hbm, sc_v, sem),
        pltpu.async_copy(e_hbm.at[pl.ds(base, CHUNK)], e_v, sem),
        pltpu.async_copy(b_hbm.at[pl.ds(base, CHUNK)], b_v, sem),
        pltpu.async_copy(at_hbm.at[pl.ds(base, CHUNK)], at_v, sem),
    )

    @pl.when(is_last)
    def _tail_in():
        tcps = (
            pltpu.async_copy(e_hbm.at[pl.ds(MAIN, TAIL)], et_v, sem),
            pltpu.async_copy(b_hbm.at[pl.ds(MAIN, TAIL)], bt_v, sem),
            pltpu.async_copy(at_hbm.at[pl.ds(MAIN, TAIL)], att_v, sem),
        )
        for cp in tcps:
            cp.wait()

    for cp in cps:
        cp.wait()

    def scale_shift(e_ref, b_ref, a_ref, o_ref, i):
        off = i * L
        b = b_ref[pl.ds(off, L)]
        a = a_ref[pl.ds(off, L)]
        m = plsc.load_gather(mt_v, [b])
        idx = m * 16 + a
        sh = plsc.load_gather(sh_v, [idx])
        sc = plsc.load_gather(sc_v, [idx])
        o_ref[pl.ds(off, L)] = e_ref[pl.ds(off, L)] * sc + sh

    @plsc.parallel_loop(0, CHUNK // L, 1, unroll=13)
    def _main(i):
        scale_shift(e_v, b_v, at_v, o_v, i)

    pltpu.sync_copy(o_v, out_hbm.at[pl.ds(base, CHUNK)])

    @pl.when(is_last)
    def _tail():
        @plsc.parallel_loop(0, TAIL // L, 1, unroll=5)
        def _t(i):
            scale_shift(et_v, bt_v, att_v, ot_v, i)
        pltpu.sync_copy(ot_v, out_hbm.at[pl.ds(MAIN, TAIL)])


_sc_call = functools.partial(
    pl.kernel,
    mesh=plsc.VectorSubcoreMesh(core_axis_name="c", subcore_axis_name="s"),
    out_type=jax.ShapeDtypeStruct((N,), jnp.float32),
    compiler_params=pltpu.CompilerParams(needs_layout_passes=False),
    scratch_types=[
        pltpu.VMEM((CHUNK,), jnp.float32),
        pltpu.VMEM((CHUNK,), jnp.int32),
        pltpu.VMEM((CHUNK,), jnp.int32),
        pltpu.VMEM((CHUNK,), jnp.float32),
        pltpu.VMEM((G,), jnp.int32),
        pltpu.VMEM((G,), jnp.float32),
        pltpu.VMEM((G,), jnp.float32),
        pltpu.VMEM((TAIL,), jnp.float32),
        pltpu.VMEM((TAIL,), jnp.int32),
        pltpu.VMEM((TAIL,), jnp.int32),
        pltpu.VMEM((TAIL,), jnp.float32),
        pltpu.SemaphoreType.DMA,
    ],
)(_sc_body)


def kernel(scaled_atomic_energy, batch, modal_type, atom_type, shift, scale):
    e = scaled_atomic_energy.reshape(-1)
    out = _sc_call(e, batch, modal_type, atom_type,
                   shift.reshape(-1), scale.reshape(-1))
    return out.reshape(-1, 1)


# speedup vs baseline: 66.7002x; 1.0784x over previous
import jax
import jax.numpy as jnp
from jax import lax
from jax.experimental import pallas as pl
from jax.experimental.pallas import tpu as pltpu
from jax.experimental.pallas import tpu_sc as plsc

N = 100000
G = 64
L = 16
NW = 32
CHUNK = 3120
MAIN = NW * CHUNK
TAIL = N - MAIN
EWIN = 3328


def _sc_body(e_hbm, b_hbm, mt_hbm, at_hbm, sh_hbm, sc_hbm, out_hbm,
             e_v, b_v, at_v, o_v, mt_v, sh_v, sc_v,
             et_v, bt_v, att_v, ot_v, sem):
    cid = lax.axis_index("c")
    sid = lax.axis_index("s")
    wid = sid * 2 + cid
    base = wid * CHUNK
    ebase = pl.multiple_of((base // 128) * 128, 128)
    eoff = base - ebase
    is_last = wid == NW - 1

    cps = (
        pltpu.async_copy(mt_hbm, mt_v, sem),
        pltpu.async_copy(sh_hbm, sh_v, sem),
        pltpu.async_copy(sc_hbm, sc_v, sem),
        pltpu.async_copy(e_hbm.at[0, pl.ds(ebase, EWIN)], e_v, sem),
        pltpu.async_copy(b_hbm.at[pl.ds(base, CHUNK)], b_v, sem),
        pltpu.async_copy(at_hbm.at[pl.ds(base, CHUNK)], at_v, sem),
    )

    @pl.when(is_last)
    def _tail_in():
        tcps = (
            pltpu.async_copy(
                e_hbm.at[0, pl.ds(pl.multiple_of(MAIN, 128), 256)],
                et_v, sem),
            pltpu.async_copy(b_hbm.at[pl.ds(MAIN, TAIL)], bt_v, sem),
            pltpu.async_copy(at_hbm.at[pl.ds(MAIN, TAIL)], att_v, sem),
        )
        for cp in tcps:
            cp.wait()

    for cp in cps:
        cp.wait()

    def scale_shift(e_ref, eo, b_ref, a_ref, o_ref, i):
        off = i * L
        b = b_ref[pl.ds(off, L)]
        a = a_ref[pl.ds(off, L)]
        m = plsc.load_gather(mt_v, [b])
        idx = m * 16 + a
        sh = plsc.load_gather(sh_v, [idx])
        sc = plsc.load_gather(sc_v, [idx])
        o_ref[pl.ds(off, L)] = e_ref[pl.ds(eo + off, L)] * sc + sh

    @plsc.parallel_loop(0, CHUNK // L, 1, unroll=13)
    def _main(i):
        scale_shift(e_v, eoff, b_v, at_v, o_v, i)

    pltpu.sync_copy(o_v, out_hbm.at[pl.ds(base, CHUNK)])

    @pl.when(is_last)
    def _tail():
        @plsc.parallel_loop(0, TAIL // L, 1, unroll=5)
        def _t(i):
            scale_shift(et_v, 0, bt_v, att_v, ot_v, i)
        pltpu.sync_copy(ot_v, out_hbm.at[pl.ds(MAIN, TAIL)])


_mesh = plsc.VectorSubcoreMesh(core_axis_name="c", subcore_axis_name="s")

_sc_call = pl.kernel(
    _sc_body,
    mesh=_mesh,
    out_type=jax.ShapeDtypeStruct((N,), jnp.float32),
    compiler_params=pltpu.CompilerParams(needs_layout_passes=False),
    scratch_types=[
        pltpu.VMEM((EWIN,), jnp.float32),
        pltpu.VMEM((CHUNK,), jnp.int32),
        pltpu.VMEM((CHUNK,), jnp.int32),
        pltpu.VMEM((CHUNK,), jnp.float32),
        pltpu.VMEM((G,), jnp.int32),
        pltpu.VMEM((G,), jnp.float32),
        pltpu.VMEM((G,), jnp.float32),
        pltpu.VMEM((256,), jnp.float32),
        pltpu.VMEM((TAIL,), jnp.int32),
        pltpu.VMEM((TAIL,), jnp.int32),
        pltpu.VMEM((TAIL,), jnp.float32),
        pltpu.SemaphoreType.DMA,
    ],
)


def kernel(scaled_atomic_energy, batch, modal_type, atom_type, shift, scale):
    e = scaled_atomic_energy.reshape(1, N)
    out = _sc_call(e, batch, modal_type, atom_type,
                   shift.reshape(-1), scale.reshape(-1))
    return out.reshape(N, 1)


# speedup vs baseline: 72.6834x; 1.0897x over previous
import jax
import jax.numpy as jnp
from jax import lax
from jax.experimental import pallas as pl
from jax.experimental.pallas import tpu as pltpu
from jax.experimental.pallas import tpu_sc as plsc

N = 100000
G = 64
L = 16
NW = 32
CHUNK = 3200
TBASE = (NW - 1) * CHUNK
TREAL = N - TBASE
TPAD = 896


def _sc_body(e_hbm, b_hbm, mt_hbm, at_hbm, tab_hbm, out_hbm,
             e_v, b_v, at_v, o_v, mt_v, tab_v, sem):
    cid = lax.axis_index("c")
    sid = lax.axis_index("s")
    wid = sid * 2 + cid
    base = wid * CHUNK
    is_last = wid == NW - 1

    tab_cps = (
        pltpu.async_copy(mt_hbm, mt_v, sem),
        pltpu.async_copy(tab_hbm, tab_v, sem),
    )

    def gathers(b, a):
        m = plsc.load_gather(mt_v, [b])
        sh = plsc.load_gather(tab_v, [m, a])
        sc = plsc.load_gather(tab_v, [m + 4, a])
        return sh, sc

    @pl.when(jnp.logical_not(is_last))
    def _main_path():
        cps = (
            pltpu.async_copy(e_hbm.at[0, pl.ds(base, CHUNK)], e_v, sem),
            pltpu.async_copy(b_hbm.at[pl.ds(base, CHUNK)],
                             b_v.at[pl.ds(0, CHUNK)], sem),
            pltpu.async_copy(at_hbm.at[pl.ds(base, CHUNK)],
                             at_v.at[pl.ds(0, CHUNK)], sem),
        )
        for cp in tab_cps + cps:
            cp.wait()

        @plsc.parallel_loop(0, CHUNK // L, 1, unroll=10)
        def _l(i):
            off = i * L
            sh, sc = gathers(b_v[pl.ds(off, L)], at_v[pl.ds(off, L)])
            o_v[pl.ds(off, L)] = e_v[pl.ds(off, L)] * sc + sh

        pltpu.sync_copy(o_v, out_hbm.at[0, pl.ds(base, CHUNK)])

    @pl.when(is_last)
    def _tail_path():
        cps = (
            pltpu.async_copy(e_hbm.at[0, pl.ds(pl.multiple_of(TBASE, 128),
                                               TPAD)],
                             e_v.at[pl.ds(0, TPAD)], sem),
            pltpu.async_copy(b_hbm.at[pl.ds(TBASE, TREAL)],
                             b_v.at[pl.ds(0, TREAL)], sem),
            pltpu.async_copy(at_hbm.at[pl.ds(TBASE, TREAL)],
                             at_v.at[pl.ds(0, TREAL)], sem),
        )
        for cp in tab_cps + cps:
            cp.wait()

        @plsc.parallel_loop(0, TPAD // L, 1, unroll=7)
        def _l(i):
            off = i * L
            b = b_v[pl.ds(off, L)] & (G - 1)
            a = at_v[pl.ds(off, L)] & 15
            sh, sc = gathers(b, a)
            o_v[pl.ds(off, L)] = e_v[pl.ds(off, L)] * sc + sh

        pltpu.sync_copy(o_v.at[pl.ds(0, TPAD)],
                        out_hbm.at[0, pl.ds(pl.multiple_of(TBASE, 128),
                                            TPAD)])


_mesh = plsc.VectorSubcoreMesh(core_axis_name="c", subcore_axis_name="s")

_sc_call = pl.kernel(
    _sc_body,
    mesh=_mesh,
    out_type=jax.ShapeDtypeStruct((1, N), jnp.float32),
    compiler_params=pltpu.CompilerParams(needs_layout_passes=False),
    scratch_types=[
        pltpu.VMEM((CHUNK,), jnp.float32),
        pltpu.VMEM((CHUNK,), jnp.int32),
        pltpu.VMEM((CHUNK,), jnp.int32),
        pltpu.VMEM((CHUNK,), jnp.float32),
        pltpu.VMEM((G,), jnp.int32),
        pltpu.VMEM((8, 16), jnp.float32),
        pltpu.SemaphoreType.DMA,
    ],
)


def kernel(scaled_atomic_energy, batch, modal_type, atom_type, shift, scale):
    e = scaled_atomic_energy.reshape(1, N)
    tab = jnp.concatenate([shift, scale], axis=0)
    out = _sc_call(e, batch, modal_type, atom_type, tab)
    return out.reshape(N, 1)


# speedup vs baseline: 73.0305x; 1.0048x over previous
import jax
import jax.numpy as jnp
from jax import lax
from jax.experimental import pallas as pl
from jax.experimental.pallas import tpu as pltpu
from jax.experimental.pallas import tpu_sc as plsc

N = 100000
G = 64
L = 16
NW = 32
CHUNK = 3200
TBASE = (NW - 1) * CHUNK
TREAL = N - TBASE
TPAD = 896


def _sc_body(e_hbm, b_hbm, mt_hbm, at_hbm, sh_hbm, sc_hbm, out_hbm,
             e_v, b_v, at_v, o_v, mt_v, sh_t, sc_t, sem):
    cid = lax.axis_index("c")
    sid = lax.axis_index("s")
    wid = sid * 2 + cid
    base = wid * CHUNK
    is_last = wid == NW - 1

    tab_cps = (
        pltpu.async_copy(mt_hbm, mt_v, sem),
        pltpu.async_copy(sh_hbm, sh_t, sem),
        pltpu.async_copy(sc_hbm, sc_t, sem),
    )

    def gathers(b, a):
        m = plsc.load_gather(mt_v, [b])
        sh = plsc.load_gather(sh_t, [m, a])
        sc = plsc.load_gather(sc_t, [m, a])
        return sh, sc

    @pl.when(jnp.logical_not(is_last))
    def _main_path():
        cps = (
            pltpu.async_copy(e_hbm.at[0, pl.ds(base, CHUNK)], e_v, sem),
            pltpu.async_copy(b_hbm.at[pl.ds(base, CHUNK)],
                             b_v.at[pl.ds(0, CHUNK)], sem),
            pltpu.async_copy(at_hbm.at[pl.ds(base, CHUNK)],
                             at_v.at[pl.ds(0, CHUNK)], sem),
        )
        for cp in tab_cps + cps:
            cp.wait()

        @plsc.parallel_loop(0, CHUNK // L, 1, unroll=10)
        def _l(i):
            off = i * L
            sh, sc = gathers(b_v[pl.ds(off, L)], at_v[pl.ds(off, L)])
            o_v[pl.ds(off, L)] = e_v[pl.ds(off, L)] * sc + sh

        pltpu.sync_copy(o_v, out_hbm.at[0, pl.ds(base, CHUNK)])

    @pl.when(is_last)
    def _tail_path():
        cps = (
            pltpu.async_copy(e_hbm.at[0, pl.ds(pl.multiple_of(TBASE, 128),
                                               TPAD)],
                             e_v.at[pl.ds(0, TPAD)], sem),
            pltpu.async_copy(b_hbm.at[pl.ds(TBASE, TREAL)],
                             b_v.at[pl.ds(0, TREAL)], sem),
            pltpu.async_copy(at_hbm.at[pl.ds(TBASE, TREAL)],
                             at_v.at[pl.ds(0, TREAL)], sem),
        )
        for cp in tab_cps + cps:
            cp.wait()

        @plsc.parallel_loop(0, TPAD // L, 1, unroll=7)
        def _l(i):
            off = i * L
            b = b_v[pl.ds(off, L)] & (G - 1)
            a = at_v[pl.ds(off, L)] & 15
            sh, sc = gathers(b, a)
            o_v[pl.ds(off, L)] = e_v[pl.ds(off, L)] * sc + sh

        pltpu.sync_copy(o_v.at[pl.ds(0, TPAD)],
                        out_hbm.at[0, pl.ds(pl.multiple_of(TBASE, 128),
                                            TPAD)])


_mesh = plsc.VectorSubcoreMesh(core_axis_name="c", subcore_axis_name="s")

_sc_call = pl.kernel(
    _sc_body,
    mesh=_mesh,
    out_type=jax.ShapeDtypeStruct((1, N), jnp.float32),
    compiler_params=pltpu.CompilerParams(needs_layout_passes=False),
    scratch_types=[
        pltpu.VMEM((CHUNK,), jnp.float32),
        pltpu.VMEM((CHUNK,), jnp.int32),
        pltpu.VMEM((CHUNK,), jnp.int32),
        pltpu.VMEM((CHUNK,), jnp.float32),
        pltpu.VMEM((G,), jnp.int32),
        pltpu.VMEM((4, 16), jnp.float32),
        pltpu.VMEM((4, 16), jnp.float32),
        pltpu.SemaphoreType.DMA,
    ],
)


def kernel(scaled_atomic_energy, batch, modal_type, atom_type, shift, scale):
    e = scaled_atomic_energy.reshape(1, N)
    out = _sc_call(e, batch, modal_type, atom_type, shift, scale)
    return out.reshape(N, 1)
